# baseline (speedup 1.0000x reference)
import jax
import jax.numpy as jnp
from jax.experimental import pallas as pl

_TI = 1024


def _nn_kernel(p1_ref, x1_ref, p2_ref, x2_ref, idx_ref, dir_ref):
    p1 = p1_ref[0]
    x1 = x1_ref[0]
    p2 = p2_ref[0]
    x2 = x2_ref[0]
    f1 = jnp.concatenate([p1, x1], axis=0)
    f2 = jnp.concatenate([p2, x2], axis=0)
    n2 = f2.shape[1]
    dots = jax.lax.dot_general(
        f1, f2, (((0,), (0,)), ((), ())), preferred_element_type=jnp.float32
    )
    sq1 = jnp.sum(f1 * f1, axis=0).reshape(_TI, 1)
    sq2 = jnp.sum(f2 * f2, axis=0, keepdims=True)
    d = -2.0 * dots + sq1 + sq2
    dmin = jnp.min(d, axis=1, keepdims=True)
    jidx = jax.lax.broadcasted_iota(jnp.int32, d.shape, 1)
    idx = jnp.min(jnp.where(d == dmin, jidx, jnp.int32(n2)), axis=1)
    onehot = (jidx == idx[:, None]).astype(jnp.float32)
    nb = jax.lax.dot_general(
        x2, onehot, (((1,), (1,)), ((), ())), preferred_element_type=jnp.float32
    )
    dir_ref[0] = nb - x1
    idx_ref[0] = idx[None, :]


def kernel(xyz1, xyz2, points1, points2):
    B, C, N1 = xyz1.shape
    N2 = xyz2.shape[2]
    D = points1.shape[1]

    idx_out, dir_out = pl.pallas_call(
        _nn_kernel,
        grid=(B, N1 // _TI),
        in_specs=[
            pl.BlockSpec((1, D, _TI), lambda b, i: (b, 0, i)),
            pl.BlockSpec((1, C, _TI), lambda b, i: (b, 0, i)),
            pl.BlockSpec((1, D, N2), lambda b, i: (b, 0, 0)),
            pl.BlockSpec((1, C, N2), lambda b, i: (b, 0, 0)),
        ],
        out_specs=[
            pl.BlockSpec((1, 1, _TI), lambda b, i: (b, 0, i)),
            pl.BlockSpec((1, C, _TI), lambda b, i: (b, 0, i)),
        ],
        out_shape=[
            jax.ShapeDtypeStruct((B, 1, N1), jnp.int32),
            jax.ShapeDtypeStruct((B, C, N1), jnp.float32),
        ],
    )(points1, xyz1, points2, xyz2)

    corres1 = jnp.broadcast_to(
        jnp.arange(N1, dtype=jnp.int32)[None, None, :], (B, 1, N1)
    )
    return (corres1, idx_out, dir_out)


# speedup vs baseline: 1.0331x; 1.0331x over previous
import jax
import jax.numpy as jnp
from jax.experimental import pallas as pl

_TI = 1024


def _nn_kernel(p1_ref, x1_ref, p2_ref, x2_ref, idx_ref, dir_ref):
    p1 = p1_ref[0]
    x1 = x1_ref[0]
    p2 = p2_ref[0]
    x2 = x2_ref[0]
    f1 = jnp.concatenate([p1, x1], axis=0)
    f2 = jnp.concatenate([p2, x2], axis=0)
    n2 = f2.shape[1]
    F = f1.shape[0]
    dots = jax.lax.dot_general(
        f1, f2, (((0,), (0,)), ((), ())), preferred_element_type=jnp.float32
    )
    sq1h = (0.5 * jnp.sum(f1 * f1, axis=0)).reshape(_TI, 1)
    sq2h = 0.5 * jnp.sum(f2 * f2, axis=0, keepdims=True)
    q = (dots - sq1h) - sq2h
    qmax = jnp.max(q, axis=1, keepdims=True)
    jidx = jax.lax.broadcasted_iota(jnp.int32, q.shape, 1)
    idx = jnp.min(jnp.where(q == qmax, jidx, jnp.int32(n2)), axis=1)
    onehot = (jidx == idx[:, None]).astype(jnp.float32)
    nb = jax.lax.dot_general(
        x2, onehot, (((1,), (1,)), ((), ())), preferred_element_type=jnp.float32
    )
    dir_ref[0] = nb - x1
    idx_ref[0] = idx[None, :]


def kernel(xyz1, xyz2, points1, points2):
    B, C, N1 = xyz1.shape
    N2 = xyz2.shape[2]
    D = points1.shape[1]

    idx_out, dir_out = pl.pallas_call(
        _nn_kernel,
        grid=(B, N1 // _TI),
        in_specs=[
            pl.BlockSpec((1, D, _TI), lambda b, i: (b, 0, i)),
            pl.BlockSpec((1, C, _TI), lambda b, i: (b, 0, i)),
            pl.BlockSpec((1, D, N2), lambda b, i: (b, 0, 0)),
            pl.BlockSpec((1, C, N2), lambda b, i: (b, 0, 0)),
        ],
        out_specs=[
            pl.BlockSpec((1, 1, _TI), lambda b, i: (b, 0, i)),
            pl.BlockSpec((1, C, _TI), lambda b, i: (b, 0, i)),
        ],
        out_shape=[
            jax.ShapeDtypeStruct((B, 1, N1), jnp.int32),
            jax.ShapeDtypeStruct((B, C, N1), jnp.float32),
        ],
    )(points1, xyz1, points2, xyz2)

    corres1 = jnp.broadcast_to(
        jnp.arange(N1, dtype=jnp.int32)[None, None, :], (B, 1, N1)
    )
    return (corres1, idx_out, dir_out)


# speedup vs baseline: 1.0410x; 1.0077x over previous
import jax
import jax.numpy as jnp
from jax.experimental import pallas as pl

_TI = 2048


def _nn_kernel(p1_ref, x1_ref, p2_ref, x2_ref, idx_ref, dir_ref):
    p1 = p1_ref[0]
    x1 = x1_ref[0]
    p2 = p2_ref[0]
    x2 = x2_ref[0]
    f1 = jnp.concatenate([p1, x1], axis=0)
    f2 = jnp.concatenate([p2, x2], axis=0)
    n2 = f2.shape[1]
    F = f1.shape[0]
    dots = jax.lax.dot_general(
        f1, f2, (((0,), (0,)), ((), ())), preferred_element_type=jnp.float32
    )
    sq1h = (0.5 * jnp.sum(f1 * f1, axis=0)).reshape(_TI, 1)
    sq2h = 0.5 * jnp.sum(f2 * f2, axis=0, keepdims=True)
    q = (dots - sq1h) - sq2h
    qmax = jnp.max(q, axis=1, keepdims=True)
    jidx = jax.lax.broadcasted_iota(jnp.int32, q.shape, 1)
    idx = jnp.min(jnp.where(q == qmax, jidx, jnp.int32(n2)), axis=1)
    onehot = (jidx == idx[:, None]).astype(jnp.float32)
    nb = jax.lax.dot_general(
        x2, onehot, (((1,), (1,)), ((), ())), preferred_element_type=jnp.float32
    )
    dir_ref[0] = nb - x1
    idx_ref[0] = idx[None, :]


def kernel(xyz1, xyz2, points1, points2):
    B, C, N1 = xyz1.shape
    N2 = xyz2.shape[2]
    D = points1.shape[1]

    idx_out, dir_out = pl.pallas_call(
        _nn_kernel,
        grid=(B, N1 // _TI),
        in_specs=[
            pl.BlockSpec((1, D, _TI), lambda b, i: (b, 0, i)),
            pl.BlockSpec((1, C, _TI), lambda b, i: (b, 0, i)),
            pl.BlockSpec((1, D, N2), lambda b, i: (b, 0, 0)),
            pl.BlockSpec((1, C, N2), lambda b, i: (b, 0, 0)),
        ],
        out_specs=[
            pl.BlockSpec((1, 1, _TI), lambda b, i: (b, 0, i)),
            pl.BlockSpec((1, C, _TI), lambda b, i: (b, 0, i)),
        ],
        out_shape=[
            jax.ShapeDtypeStruct((B, 1, N1), jnp.int32),
            jax.ShapeDtypeStruct((B, C, N1), jnp.float32),
        ],
    )(points1, xyz1, points2, xyz2)

    corres1 = jnp.broadcast_to(
        jnp.arange(N1, dtype=jnp.int32)[None, None, :], (B, 1, N1)
    )
    return (corres1, idx_out, dir_out)
